# Initial kernel scaffold; baseline (speedup 1.0000x reference)
import jax
import jax.numpy as jnp
from jax.experimental import pallas as pl


def kernel(stories, table, mask):
    raise NotImplementedError("write your pallas kernel here")


# speedup vs baseline: 1.5522x; 1.5522x over previous
import jax
import jax.numpy as jnp
from jax import lax
from jax.experimental import pallas as pl
from jax.experimental.pallas import tpu as pltpu
from jax.experimental.pallas import tpu_sc as plsc

NC = 2
NS = 16
NW = NC * NS

IDX_PER_STREAM = 128


def _make_sc_call(B, S, L, D, V):
    SENT = B * S
    assert SENT % NW == 0
    sent_per_w = SENT // NW
    chunk = (IDX_PER_STREAM * L // _gcd(IDX_PER_STREAM, L)) // L
    assert (chunk * L) % IDX_PER_STREAM == 0
    n_streams = chunk * L // IDX_PER_STREAM
    assert sent_per_w % chunk == 0
    n_chunks = sent_per_w // chunk
    idx_per_chunk = chunk * L

    mesh = plsc.VectorSubcoreMesh(core_axis_name="c", subcore_axis_name="s")

    @pl.kernel(
        out_type=jax.ShapeDtypeStruct((SENT, D), jnp.float32),
        mesh=mesh,
        compiler_params=pltpu.CompilerParams(use_tc_tiling_on_sc=False),
        scratch_types=[
            pltpu.VMEM((idx_per_chunk,), jnp.int32),
            pltpu.VMEM((chunk * L, D), jnp.float32),
            pltpu.VMEM((chunk, D), jnp.float32),
            pltpu.VMEM((L, D), jnp.float32),
            pltpu.SemaphoreType.DMA,
        ],
    )
    def sc_call(table_hbm, idx_hbm, mask_hbm, out_hbm,
                idx_v, rows_v, out_v, mask_v, sem):
        wid = lax.axis_index("s") * NC + lax.axis_index("c")
        pltpu.sync_copy(mask_hbm, mask_v)

        @pl.loop(0, n_chunks)
        def _chunk(c):
            sent0 = wid * sent_per_w + c * chunk
            i0 = sent0 * L
            pltpu.sync_copy(idx_hbm.at[pl.ds(i0, idx_per_chunk)], idx_v)
            copies = [
                pltpu.async_copy(
                    table_hbm.at[idx_v.at[pl.ds(j * IDX_PER_STREAM,
                                                IDX_PER_STREAM)]],
                    rows_v.at[pl.ds(j * IDX_PER_STREAM, IDX_PER_STREAM)],
                    sem)
                for j in range(n_streams)
            ]
            for cp in copies:
                cp.wait()

            for dc in range(D // 16):
                dsl = pl.ds(dc * 16, 16)
                m = [mask_v[l, dsl] for l in range(L)]

                @pl.loop(0, chunk)
                def _sent(si):
                    base = si * L
                    acc = rows_v[base, dsl] * m[0]
                    for l in range(1, L):
                        acc = acc + rows_v[base + l, dsl] * m[l]
                    out_v[si, dsl] = acc

            pltpu.sync_copy(out_v, out_hbm.at[pl.ds(sent0, chunk)])

    return sc_call


def _gcd(a, b):
    while b:
        a, b = b, a % b
    return a


def kernel(stories, table, mask):
    B, S, L = stories.shape
    V, D = table.shape
    idx_flat = stories.astype(jnp.int32).reshape(-1)
    sc_call = _make_sc_call(B, S, L, D, V)
    out = sc_call(table, idx_flat, mask.astype(jnp.float32))
    return out.reshape(B, S, D)


# speedup vs baseline: 1.7973x; 1.1579x over previous
import jax
import jax.numpy as jnp
from jax import lax
from jax.experimental import pallas as pl
from jax.experimental.pallas import tpu as pltpu
from jax.experimental.pallas import tpu_sc as plsc

NC = 2
NS = 16
NW = NC * NS

IDX_PER_STREAM = 128
NBUF = 2


def _gcd(a, b):
    while b:
        a, b = b, a % b
    return a


def _make_sc_call(B, S, L, D, V):
    SENT = B * S
    assert SENT % NW == 0
    sent_per_w = SENT // NW
    chunk = IDX_PER_STREAM // _gcd(IDX_PER_STREAM, L)
    ipc = chunk * L
    n_streams = ipc // IDX_PER_STREAM
    assert sent_per_w % (chunk * NBUF) == 0
    n_chunks = sent_per_w // chunk

    mesh = plsc.VectorSubcoreMesh(core_axis_name="c", subcore_axis_name="s")

    @pl.kernel(
        out_type=jax.ShapeDtypeStruct((SENT, D), jnp.float32),
        mesh=mesh,
        compiler_params=pltpu.CompilerParams(use_tc_tiling_on_sc=False),
        scratch_types=[
            pltpu.VMEM((NBUF, ipc), jnp.int32),
            pltpu.VMEM((NBUF, ipc, D), jnp.float32),
            pltpu.VMEM((chunk, D), jnp.float32),
            pltpu.VMEM((L, D), jnp.float32),
            pltpu.SemaphoreType.DMA,
            pltpu.SemaphoreType.DMA,
            pltpu.SemaphoreType.DMA,
            pltpu.SemaphoreType.DMA,
        ],
    )
    def sc_call(table_hbm, idx_hbm, mask_hbm, out_hbm,
                idx_v, rows_v, out_v, mask_v, sg0, sg1, si0, si1):
        wid = lax.axis_index("s") * NC + lax.axis_index("c")
        pltpu.sync_copy(mask_hbm, mask_v)
        sent_base = wid * sent_per_w
        idx_base = sent_base * L
        sems_g = [sg0, sg1]
        sems_i = [si0, si1]

        def idx_src(c):
            return idx_hbm.at[pl.ds(idx_base + c * ipc, ipc)]

        def fire_gathers(b):
            for j in range(n_streams):
                js = pl.ds(j * IDX_PER_STREAM, IDX_PER_STREAM)
                pltpu.async_copy(table_hbm.at[idx_v.at[b, js]],
                                 rows_v.at[b, js], sems_g[b])

        def drain_gathers(b):
            for j in range(n_streams):
                js = pl.ds(j * IDX_PER_STREAM, IDX_PER_STREAM)
                pltpu.make_async_copy(table_hbm.at[idx_v.at[b, js]],
                                      rows_v.at[b, js], sems_g[b]).wait()

        def compute(c, b):
            for dc in range(D // 16):
                dsl = pl.ds(dc * 16, 16)
                m = [mask_v[l, dsl] for l in range(L)]

                @pl.loop(0, chunk, unroll=2)
                def _sent(s):
                    base = s * L
                    acc = rows_v[b, base, dsl] * m[0]
                    for l in range(1, L):
                        acc = acc + rows_v[b, base + l, dsl] * m[l]
                    out_v[s, dsl] = acc

            pltpu.sync_copy(out_v,
                            out_hbm.at[pl.ds(sent_base + c * chunk, chunk)])

        for b in range(NBUF):
            pltpu.sync_copy(idx_src(b), idx_v.at[b])
            fire_gathers(b)

        @pl.loop(0, n_chunks, step=NBUF)
        def _chunks(c):
            for b in range(NBUF):
                cc = c + b
                nxt = cc + NBUF
                drain_gathers(b)

                @pl.when(nxt < n_chunks)
                def _prefetch_idx():
                    pltpu.async_copy(idx_src(nxt), idx_v.at[b], sems_i[b])

                compute(cc, b)

                @pl.when(nxt < n_chunks)
                def _fire_next():
                    pltpu.make_async_copy(idx_src(nxt), idx_v.at[b],
                                          sems_i[b]).wait()
                    fire_gathers(b)

    return sc_call


def kernel(stories, table, mask):
    B, S, L = stories.shape
    V, D = table.shape
    idx_flat = stories.astype(jnp.int32).reshape(-1)
    sc_call = _make_sc_call(B, S, L, D, V)
    out = sc_call(table, idx_flat, mask.astype(jnp.float32))
    return out.reshape(B, S, D)
